# baseline (speedup 1.0000x reference)
import functools

import jax
import jax.numpy as jnp
from jax import lax
from jax.experimental import pallas as pl
from jax.experimental.pallas import tpu as pltpu
from jax.experimental.pallas import tpu_sc as plsc

_TB = 256
_NW = 32
_S = 1664


def _tc_kernel(lens_ref, x_ref, tgt_ref, out_ref, c_ref, m_ref):
    j = pl.program_id(0)
    nj = pl.num_programs(0)
    x = x_ref[...]
    TB, B, V = x.shape
    rowmax = jnp.max(x, axis=-1, keepdims=True)
    idx = lax.broadcasted_iota(jnp.int32, x.shape, 2)
    pred = jnp.min(jnp.where(x == rowmax, idx, V), axis=-1)
    tgt = tgt_ref[0]
    t_idx = lax.broadcasted_iota(jnp.int32, (TB, B), 0) + j * TB
    b_idx = lax.broadcasted_iota(jnp.int32, (TB, B), 1)
    lens_v = jnp.zeros((TB, B), jnp.int32)
    for b in range(B):
        lens_v = jnp.where(b_idx == b, lens_ref[b] + 1, lens_v)
    mask = t_idx < jnp.minimum(lens_v, _S)
    c_part = jnp.sum(jnp.where(mask & (pred == tgt), 1.0, 0.0))
    m_part = jnp.sum(jnp.where(mask, 1.0, 0.0))

    @pl.when(j == 0)
    def _init():
        c_ref[0] = 0.0
        m_ref[0] = 0.0

    c_ref[0] += c_part
    m_ref[0] += m_part

    @pl.when(j == nj - 1)
    def _fin():
        out_ref[0] = c_ref[0]
        out_ref[1] = m_ref[0]


def _tc_counts(outputs, tgt, tokens_lens):
    T, B, V = outputs.shape
    nj = _S // _TB
    tgt3 = tgt.reshape(T // _TB, _TB, B)
    grid_spec = pltpu.PrefetchScalarGridSpec(
        num_scalar_prefetch=1,
        grid=(nj,),
        in_specs=[
            pl.BlockSpec((_TB, B, V), lambda j, lens: (j, 0, 0)),
            pl.BlockSpec((1, _TB, B), lambda j, lens: (j, 0, 0)),
        ],
        out_specs=pl.BlockSpec(memory_space=pltpu.SMEM),
        scratch_shapes=[
            pltpu.SMEM((1,), jnp.float32),
            pltpu.SMEM((1,), jnp.float32),
        ],
    )
    return pl.pallas_call(
        _tc_kernel,
        grid_spec=grid_spec,
        out_shape=jax.ShapeDtypeStruct((2,), jnp.float32),
        compiler_params=pltpu.CompilerParams(
            dimension_semantics=("arbitrary",),
        ),
    )(tokens_lens, outputs, tgt3)


def _all_lanes(x, op):
    io = lax.broadcasted_iota(jnp.int32, (16,), 0)
    dnums = lax.GatherDimensionNumbers(
        offset_dims=(), collapsed_slice_dims=(0,), start_index_map=(0,)
    )
    for sh in (8, 4, 2, 1):
        idx = ((io + sh) & 15)[:, None]
        rot = lax.gather(
            x, idx, dnums, (1,),
            mode=lax.GatherScatterMode.PROMISE_IN_BOUNDS,
        )
        x = op(x, rot)
    return x


def _sc_counts(x2d, tgt_flat, lens_pad, T, B, V):
    nchunk = V // 16
    T16 = T + 16

    mesh = plsc.VectorSubcoreMesh(core_axis_name="c", subcore_axis_name="s")

    @functools.partial(
        pl.kernel,
        mesh=mesh,
        out_type=jax.ShapeDtypeStruct((_NW, 2, 16), jnp.int32),
        scratch_types=[
            pltpu.VMEM((16, V), jnp.float32),
            pltpu.VMEM((16, V), jnp.float32),
            pltpu.VMEM((B * (T + 16),), jnp.int32),
            pltpu.VMEM((32,), jnp.int32),
            pltpu.VMEM((2, 16), jnp.int32),
            pltpu.SemaphoreType.DMA,
            pltpu.SemaphoreType.DMA,
        ],
    )
    def sck(x_hbm, tgtf_hbm, lens_hbm, out_hbm,
            rows_a, rows_b, tgtbuf, lensbuf, cntbuf, sem_a, sem_b):
        wid = lax.axis_index("s") * 2 + lax.axis_index("c")
        io = lax.broadcasted_iota(jnp.int32, (16,), 0)

        pltpu.sync_copy(lens_hbm, lensbuf)
        len_c, s_c, off = [], [], [jnp.int32(0)]
        for c in range(B):
            lc = lensbuf[pl.ds(c, 16)][0] + 1
            sc = jnp.minimum(jnp.int32(_S), lc)
            tail_pad = lax.div(lc - sc + 15, 16) * 16
            len_c.append(lc)
            s_c.append(sc)
            off.append(off[-1] + tail_pad)
        gtot = lax.div(off[B], 16)
        per_w = lax.div(gtot + (_NW - 1), _NW)
        g0 = wid * per_w
        g1 = jnp.minimum(g0 + per_w, gtot)
        ngw = jnp.maximum(g1 - g0, 0)

        def group_info(g_loc):
            g16 = (g0 + g_loc) * 16
            c = jnp.int32(0)
            for k in range(1, B):
                c = c + jnp.where(g16 >= off[k], 1, 0)
            lsel = jnp.int32(0)
            ssel = jnp.int32(0)
            osel = jnp.int32(0)
            for k in range(B):
                lsel = jnp.where(c == k, len_c[k], lsel)
                ssel = jnp.where(c == k, s_c[k], ssel)
                osel = jnp.where(c == k, off[k], osel)
            tbase = ssel + (g16 - osel)
            return c, tbase, lsel

        def start(g_loc, buf, sem):
            c, tbase, lsel = group_info(jnp.minimum(g_loc, ngw - 1))
            ridx = jnp.minimum(tbase + io, lsel - 1) * B + c
            pltpu.make_async_copy(x_hbm.at[ridx], buf, sem).start()

        def wait(buf, sem):
            c, tbase, lsel = group_info(jnp.int32(0))
            ridx = jnp.minimum(tbase + io, lsel - 1) * B + c
            pltpu.make_async_copy(x_hbm.at[ridx], buf, sem).wait()

        def merge(ma, ba, mb, bb):
            m = jnp.maximum(ma, mb)
            bsel = jnp.where(mb > ma, bb, ba)
            btie = jnp.minimum(ba, bb)
            return m, jnp.where(ma == mb, btie, bsel)

        def compute(g_loc, rows_v, carry):
            cc, vc = carry
            c, tbase, lsel = group_info(jnp.minimum(g_loc, ngw - 1))
            tvec = tbase + io
            preds = jnp.zeros((16,), jnp.int32)
            for gr in range(16):
                def chunk_body(jj, carry2):
                    m0, b0, m1, b1, m2, b2, m3, b3 = carry2
                    base = jj * 4
                    v0 = rows_v[gr, pl.ds((base + 0) * 16, 16)]
                    v1 = rows_v[gr, pl.ds((base + 1) * 16, 16)]
                    v2 = rows_v[gr, pl.ds((base + 2) * 16, 16)]
                    v3 = rows_v[gr, pl.ds((base + 3) * 16, 16)]
                    b0 = jnp.where(v0 > m0, base + 0, b0)
                    m0 = jnp.maximum(m0, v0)
                    b1 = jnp.where(v1 > m1, base + 1, b1)
                    m1 = jnp.maximum(m1, v1)
                    b2 = jnp.where(v2 > m2, base + 2, b2)
                    m2 = jnp.maximum(m2, v2)
                    b3 = jnp.where(v3 > m3, base + 3, b3)
                    m3 = jnp.maximum(m3, v3)
                    return m0, b0, m1, b1, m2, b2, m3, b3
                ninf = jnp.full((16,), -jnp.inf, jnp.float32)
                zi = jnp.zeros((16,), jnp.int32)
                m0, b0, m1, b1, m2, b2, m3, b3 = lax.fori_loop(
                    0, nchunk // 4, chunk_body,
                    (ninf, zi, ninf, zi, ninf, zi, ninf, zi),
                )
                m0, b0 = merge(m0, b0, m1, b1)
                m2, b2 = merge(m2, b2, m3, b3)
                m, bi = merge(m0, b0, m2, b2)
                rm = _all_lanes(m, jnp.maximum)
                cand = jnp.where(
                    m == rm, (bi * 16 + io).astype(jnp.float32), float(V)
                )
                p = _all_lanes(cand, jnp.minimum).astype(jnp.int32)
                preds = jnp.where(io == gr, p, preds)
            tg = tgtbuf[pl.ds(c * T16 + tbase, 16)]
            l_eff = jnp.where(g_loc < ngw, lsel, jnp.int32(-1))
            valid = tvec < l_eff
            cc = cc + jnp.where(valid & (preds == tg), 1, 0)
            vc = vc + jnp.where(valid, 1, 0)
            return cc, vc

        z = jnp.zeros((16,), jnp.int32)

        @pl.when(ngw > 0)
        def _work():
            pltpu.sync_copy(tgtf_hbm, tgtbuf)
            start(jnp.int32(0), rows_a, sem_a)
            start(jnp.int32(1), rows_b, sem_b)
            npairs = lax.div(ngw + 1, 2)

            def pbody(i, carry):
                g = 2 * i
                wait(rows_a, sem_a)
                carry = compute(g, rows_a, carry)
                start(g + 2, rows_a, sem_a)
                wait(rows_b, sem_b)
                carry = compute(g + 1, rows_b, carry)
                start(g + 3, rows_b, sem_b)
                return carry

            cc2, vc2 = lax.fori_loop(0, npairs, pbody, (z, z))
            wait(rows_a, sem_a)
            wait(rows_b, sem_b)
            cntbuf[0] = cc2
            cntbuf[1] = vc2

        @pl.when(ngw == 0)
        def _idle():
            cntbuf[0] = z
            cntbuf[1] = z

        pltpu.sync_copy(cntbuf, out_hbm.at[wid])

    return sck(x2d, tgt_flat, lens_pad)


def kernel(outputs, tokens, tokens_lens):
    T, B, V = outputs.shape
    lens_pad = jnp.pad(tokens_lens.astype(jnp.int32), (0, 24))
    tgt = jnp.concatenate([tokens[1:], tokens[-1:]], axis=0)
    tgt_flat = jnp.pad(tgt.T, ((0, 0), (0, 16))).reshape(B * (T + 16))
    x2d = outputs.reshape(T * B, V)

    sc = _sc_counts(x2d, tgt_flat, lens_pad, T, B, V)
    tc = _tc_counts(outputs, tgt, tokens_lens)

    scs = jnp.sum(sc, axis=(0, 2)).astype(jnp.float32)
    return (tc[0] + scs[0]) / (tc[1] + scs[1])
